# baseline (speedup 1.0000x reference)
import functools

import jax
import jax.numpy as jnp
from jax import lax
from jax.experimental import pallas as pl
from jax.experimental.pallas import tpu as pltpu
from jax.experimental.pallas import tpu_sc as plsc

_B, _S, _D = 4, 8192, 1024
_SPLIT = 3584
_NC, _NS = 2, 16
_NW = _NC * _NS
_RPW = _SPLIT // _NW
_CR = 8
_NCHUNK = _RPW // _CR
_RING = 3


def _sc_add_body(in_hbm, pos_hbm, out_hbm, in_v, pos_v, sem_in, sem_pos,
                 sem_out):
    wid = lax.axis_index("s") * _NC + lax.axis_index("c")
    pos_base = wid * _RPW

    def start_load(g, sl):
        r0 = pos_base + g * _CR
        pltpu.async_copy(pos_hbm.at[pl.ds(r0, _CR), :], pos_v[sl],
                         sem_pos[sl])
        for b in range(_B):
            pltpu.async_copy(in_hbm.at[pl.ds(b * _S + r0, _CR), :],
                             in_v[b][sl], sem_in[sl])

    def wait_load(g, sl):
        r0 = pos_base + g * _CR
        pltpu.make_async_copy(pos_hbm.at[pl.ds(r0, _CR), :], pos_v[sl],
                              sem_pos[sl]).wait()
        for b in range(_B):
            pltpu.make_async_copy(in_hbm.at[pl.ds(b * _S + r0, _CR), :],
                                  in_v[b][sl], sem_in[sl]).wait()

    def start_store(g, sl):
        r0 = pos_base + g * _CR
        for b in range(_B):
            pltpu.async_copy(in_v[b][sl],
                             out_hbm.at[pl.ds(b * _SPLIT + r0, _CR), :],
                             sem_out[sl])

    def wait_store(g, sl):
        r0 = pos_base + g * _CR
        for b in range(_B):
            pltpu.make_async_copy(in_v[b][sl],
                                  out_hbm.at[pl.ds(b * _SPLIT + r0, _CR), :],
                                  sem_out[sl]).wait()

    def compute(sl):
        @plsc.parallel_loop(0, _CR * _D // 16, step=1, unroll=8)
        def _add(k):
            r = k >> 6
            cs = pl.ds((k & 63) * 16, 16)
            pv = pos_v[sl][r, cs]
            for b in range(_B):
                in_v[b][sl][r, cs] = in_v[b][sl][r, cs] + pv

    def step(g, sl, tail=False):
        wait_load(g, sl)
        compute(sl)
        start_store(g, sl)
        if tail:
            wait_store(g - 1, (sl - 1) % _RING)
        else:
            @pl.when(g >= 1)
            def _():
                wait_store(g - 1, (sl - 1) % _RING)

            @pl.when(g + 2 < _NCHUNK)
            def _():
                start_load(g + 2, (sl + 2) % _RING)

    start_load(0, 0)
    start_load(1, 1)
    n_main = (_NCHUNK // _RING) * _RING

    def triple(t, _):
        for p in range(_RING):
            step(t * _RING + p, p)
        return 0

    lax.fori_loop(0, n_main // _RING, triple, 0)
    for g in range(n_main, _NCHUNK):
        step(g, g % _RING, tail=True)
    wait_store(_NCHUNK - 1, (_NCHUNK - 1) % _RING)


@functools.partial(
    pl.kernel,
    mesh=plsc.VectorSubcoreMesh(core_axis_name="c", subcore_axis_name="s"),
    out_type=jax.ShapeDtypeStruct((_B * _SPLIT, _D), jnp.float32),
    scratch_types=(
        [pltpu.VMEM((_CR, _D), jnp.float32)] * (_RING * (_B + 1))
        + [pltpu.SemaphoreType.DMA] * (3 * _RING)
    ),
)
def _sc_add(in_hbm, pos_hbm, out_hbm,
            i00, i01, i02, i10, i11, i12, i20, i21, i22, i30, i31, i32,
            p0, p1, p2,
            si0, si1, si2, sp0, sp1, sp2, so0, so1, so2):
    _sc_add_body(
        in_hbm, pos_hbm, out_hbm,
        [[i00, i01, i02], [i10, i11, i12], [i20, i21, i22], [i30, i31, i32]],
        [p0, p1, p2],
        [si0, si1, si2], [sp0, sp1, sp2], [so0, so1, so2])


_TBS = 512
_TOFF = _SPLIT // _TBS


def _tc_add_kernel(x_ref, p_ref, o_ref):
    o_ref[...] = x_ref[...] + p_ref[...]


def _tc_add(inputs, pos):
    B, S, D = inputs.shape
    grid = ((S - _SPLIT) // _TBS,)
    return pl.pallas_call(
        _tc_add_kernel,
        grid=grid,
        in_specs=[
            pl.BlockSpec((B, _TBS, D), lambda i: (0, i + _TOFF, 0)),
            pl.BlockSpec((1, _TBS, D), lambda i: (0, i + _TOFF, 0)),
        ],
        out_specs=pl.BlockSpec((B, _TBS, D), lambda i: (0, i + _TOFF, 0)),
        out_shape=jax.ShapeDtypeStruct((B, S, D), inputs.dtype),
    )(inputs, pos)


def kernel(inputs, pos_table):
    B, S, D = inputs.shape
    sc_out = _sc_add(inputs.reshape(B * S, D), pos_table[:_SPLIT])
    tc_out = _tc_add(inputs, pos_table[:S][None])
    return lax.dynamic_update_slice(tc_out, sc_out.reshape(B, _SPLIT, D),
                                    (0, 0, 0))


# speedup vs baseline: 1.3045x; 1.3045x over previous
import functools

import jax
import jax.numpy as jnp
from jax import lax
from jax.experimental import pallas as pl
from jax.experimental.pallas import tpu as pltpu
from jax.experimental.pallas import tpu_sc as plsc

_B, _S, _D = 4, 8192, 1024
_NC, _NS = 2, 16
_NW = _NC * _NS
_RPW = _S // _NW
_CR = 8
_NCHUNK = _RPW // _CR
_RING = 3


def _sc_add_body(in_hbm, pos_hbm, out_hbm, in_v, pos_v, sem_in, sem_pos,
                 sem_out):
    wid = lax.axis_index("s") * _NC + lax.axis_index("c")
    pos_base = wid * _RPW

    def start_load(g, sl):
        r0 = pos_base + g * _CR
        pltpu.async_copy(pos_hbm.at[pl.ds(r0, _CR), :], pos_v[sl],
                         sem_pos[sl])
        for b in range(_B):
            pltpu.async_copy(in_hbm.at[pl.ds(b * _S + r0, _CR), :],
                             in_v[b][sl], sem_in[sl])

    def wait_load(g, sl):
        r0 = pos_base + g * _CR
        pltpu.make_async_copy(pos_hbm.at[pl.ds(r0, _CR), :], pos_v[sl],
                              sem_pos[sl]).wait()
        for b in range(_B):
            pltpu.make_async_copy(in_hbm.at[pl.ds(b * _S + r0, _CR), :],
                                  in_v[b][sl], sem_in[sl]).wait()

    def start_store(g, sl):
        r0 = pos_base + g * _CR
        for b in range(_B):
            pltpu.async_copy(in_v[b][sl],
                             out_hbm.at[pl.ds(b * _S + r0, _CR), :],
                             sem_out[sl])

    def wait_store(g, sl):
        r0 = pos_base + g * _CR
        for b in range(_B):
            pltpu.make_async_copy(in_v[b][sl],
                                  out_hbm.at[pl.ds(b * _S + r0, _CR), :],
                                  sem_out[sl]).wait()

    def compute(sl):
        @plsc.parallel_loop(0, _CR * _D // 16, step=1, unroll=8)
        def _add(k):
            r = k >> 6
            cs = pl.ds((k & 63) * 16, 16)
            pv = pos_v[sl][r, cs]
            for b in range(_B):
                in_v[b][sl][r, cs] = in_v[b][sl][r, cs] + pv

    def step(g, sl, tail=False):
        wait_load(g, sl)
        compute(sl)
        start_store(g, sl)
        if tail:
            wait_store(g - 1, (sl - 1) % _RING)
        else:
            @pl.when(g >= 1)
            def _():
                wait_store(g - 1, (sl - 1) % _RING)

            @pl.when(g + 2 < _NCHUNK)
            def _():
                start_load(g + 2, (sl + 2) % _RING)

    start_load(0, 0)
    start_load(1, 1)
    n_main = (_NCHUNK // _RING) * _RING

    def triple(t, _):
        for p in range(_RING):
            step(t * _RING + p, p)
        return 0

    lax.fori_loop(0, n_main // _RING, triple, 0)
    for g in range(n_main, _NCHUNK):
        step(g, g % _RING, tail=True)
    wait_store(_NCHUNK - 1, (_NCHUNK - 1) % _RING)


@functools.partial(
    pl.kernel,
    mesh=plsc.VectorSubcoreMesh(core_axis_name="c", subcore_axis_name="s"),
    out_type=jax.ShapeDtypeStruct((_B * _S, _D), jnp.float32),
    scratch_types=(
        [pltpu.VMEM((_CR, _D), jnp.float32)] * (_RING * (_B + 1))
        + [pltpu.SemaphoreType.DMA] * (3 * _RING)
    ),
)
def _sc_add(in_hbm, pos_hbm, out_hbm,
            i00, i01, i02, i10, i11, i12, i20, i21, i22, i30, i31, i32,
            p0, p1, p2,
            si0, si1, si2, sp0, sp1, sp2, so0, so1, so2):
    _sc_add_body(
        in_hbm, pos_hbm, out_hbm,
        [[i00, i01, i02], [i10, i11, i12], [i20, i21, i22], [i30, i31, i32]],
        [p0, p1, p2],
        [si0, si1, si2], [sp0, sp1, sp2], [so0, so1, so2])


def kernel(inputs, pos_table):
    B, S, D = inputs.shape
    out = _sc_add(inputs.reshape(B * S, D), pos_table[:S])
    return out.reshape(B, S, D)
